# baseline (speedup 1.0000x reference)
import functools

import jax
import jax.numpy as jnp
from jax import lax
from jax.experimental import pallas as pl
from jax.experimental.pallas import tpu as pltpu
from jax.experimental.pallas import tpu_sc as plsc

D = 64


@functools.lru_cache(maxsize=None)
def _build(T: int, V: int):
    info = plsc.get_sparse_core_info()
    NC, NS = info.num_cores, info.num_subcores
    NW = NC * NS
    assert T % NW == 0
    b_per_w = T // NW
    C = 800
    assert b_per_w % C == 0
    n_chunks = b_per_w // C

    mesh = plsc.VectorSubcoreMesh(core_axis_name="c", subcore_axis_name="s")

    @functools.partial(
        pl.kernel,
        mesh=mesh,
        compiler_params=pltpu.CompilerParams(use_tc_tiling_on_sc=False),
        out_type=jax.ShapeDtypeStruct((T, D), jnp.float32),
        scratch_types=[
            pltpu.VMEM((b_per_w,), jnp.int32),
            pltpu.VMEM((2, C, D), jnp.float32),
            pltpu.SemaphoreType.DMA,
            pltpu.SemaphoreType.DMA,
        ],
    )
    def gather_kernel(table_hbm, idx_hbm, out_hbm, idx_v, rows_v, gsem, ssem):
        wid = lax.axis_index("s") * NC + lax.axis_index("c")
        base = wid * b_per_w
        pltpu.sync_copy(idx_hbm.at[pl.ds(base, b_per_w)], idx_v)
        gathers = [None] * n_chunks
        scatters = [None] * n_chunks
        gathers[0] = pltpu.async_copy(
            table_hbm.at[idx_v.at[pl.ds(0, C)]], rows_v.at[0], gsem
        )
        for j in range(n_chunks):
            if j + 1 < n_chunks:
                if j >= 1:
                    scatters[j - 1].wait()
                gathers[j + 1] = pltpu.async_copy(
                    table_hbm.at[idx_v.at[pl.ds((j + 1) * C, C)]],
                    rows_v.at[(j + 1) % 2],
                    gsem,
                )
            gathers[j].wait()
            scatters[j] = pltpu.async_copy(
                rows_v.at[j % 2], out_hbm.at[pl.ds(base + j * C, C)], ssem
            )
        scatters[n_chunks - 2].wait()
        scatters[n_chunks - 1].wait()

    return gather_kernel


def kernel(caption, table):
    B, L = caption.shape
    T = B * L
    idx = caption.reshape(T).astype(jnp.int32)
    out = _build(T, table.shape[0])(table, idx)
    return out.reshape(B, L, D)


# speedup vs baseline: 1.0188x; 1.0188x over previous
import functools

import jax
import jax.numpy as jnp
from jax import lax
from jax.experimental import pallas as pl
from jax.experimental.pallas import tpu as pltpu
from jax.experimental.pallas import tpu_sc as plsc

D = 64


@functools.lru_cache(maxsize=None)
def _build(T: int, V: int):
    info = plsc.get_sparse_core_info()
    NC, NS = info.num_cores, info.num_subcores
    NW = NC * NS
    assert T % NW == 0
    b_per_w = T // NW
    C = 800
    assert b_per_w % C == 0
    n_chunks = b_per_w // C

    mesh = plsc.VectorSubcoreMesh(core_axis_name="c", subcore_axis_name="s")

    @functools.partial(
        pl.kernel,
        mesh=mesh,
        compiler_params=pltpu.CompilerParams(use_tc_tiling_on_sc=False),
        out_type=jax.ShapeDtypeStruct((T, D), jnp.float32),
        scratch_types=[
            pltpu.VMEM((b_per_w,), jnp.int32),
            pltpu.VMEM((2, C, D), jnp.float32),
            pltpu.SemaphoreType.DMA,
            pltpu.SemaphoreType.DMA,
        ],
    )
    def gather_kernel(table_hbm, idx_hbm, out_hbm, idx_v, rows_v, gsem, ssem):
        wid = lax.axis_index("s") * NC + lax.axis_index("c")
        base = wid * b_per_w
        pltpu.sync_copy(idx_hbm.at[pl.ds(base, b_per_w)], idx_v)
        gathers = [None] * n_chunks
        scatters = [None] * n_chunks
        gathers[0] = pltpu.async_copy(
            table_hbm.at[idx_v.at[pl.ds(0, C)]], rows_v.at[0], gsem
        )
        for j in range(n_chunks):
            if j + 1 < n_chunks:
                if j >= 1:
                    scatters[j - 1].wait()
                gathers[j + 1] = pltpu.async_copy(
                    table_hbm.at[idx_v.at[pl.ds((j + 1) * C, C)]],
                    rows_v.at[(j + 1) % 2],
                    gsem,
                )
            gathers[j].wait()
            scatters[j] = pltpu.async_copy(
                rows_v.at[j % 2], out_hbm.at[pl.ds(base + j * C, C)], ssem
            )
        scatters[n_chunks - 2].wait()
        scatters[n_chunks - 1].wait()

    return gather_kernel


def kernel(caption, table):
    B, L = caption.shape
    T = B * L
    idx = jnp.swapaxes(caption, 0, 1).reshape(T).astype(jnp.int32)
    out = _build(T, table.shape[0])(table, idx)
    return jnp.swapaxes(out.reshape(L, B, D), 0, 1)


# speedup vs baseline: 1.1784x; 1.1567x over previous
import functools

import jax
import jax.numpy as jnp
from jax import lax
from jax.experimental import pallas as pl
from jax.experimental.pallas import tpu as pltpu
from jax.experimental.pallas import tpu_sc as plsc

D = 64
DP = 128


@functools.lru_cache(maxsize=None)
def _build(T: int, V: int):
    info = plsc.get_sparse_core_info()
    NC, NS = info.num_cores, info.num_subcores
    NW = NC * NS
    assert T % NW == 0
    b_per_w = T // NW
    C = 400
    assert b_per_w % C == 0
    n_chunks = b_per_w // C

    mesh = plsc.VectorSubcoreMesh(core_axis_name="c", subcore_axis_name="s")

    @functools.partial(
        pl.kernel,
        mesh=mesh,
        compiler_params=pltpu.CompilerParams(use_tc_tiling_on_sc=True),
        out_type=jax.ShapeDtypeStruct((T, DP), jnp.float32),
        scratch_types=[
            pltpu.VMEM((b_per_w,), jnp.int32),
            pltpu.VMEM((2, C, DP), jnp.float32),
            pltpu.SemaphoreType.DMA,
            pltpu.SemaphoreType.DMA,
        ],
    )
    def gather_kernel(table_hbm, idx_hbm, out_hbm, idx_v, rows_v, gsem, ssem):
        wid = lax.axis_index("s") * NC + lax.axis_index("c")
        base = wid * b_per_w
        pltpu.sync_copy(idx_hbm.at[pl.ds(base, b_per_w)], idx_v)
        gathers = [None] * n_chunks
        scatters = [None] * n_chunks
        gathers[0] = pltpu.async_copy(
            table_hbm.at[idx_v.at[pl.ds(0, C)]], rows_v.at[0], gsem
        )
        for j in range(n_chunks):
            if j + 1 < n_chunks:
                if j >= 1:
                    scatters[j - 1].wait()
                gathers[j + 1] = pltpu.async_copy(
                    table_hbm.at[idx_v.at[pl.ds((j + 1) * C, C)]],
                    rows_v.at[(j + 1) % 2],
                    gsem,
                )
            gathers[j].wait()
            scatters[j] = pltpu.async_copy(
                rows_v.at[j % 2], out_hbm.at[pl.ds(base + j * C, C)], ssem
            )
        scatters[n_chunks - 2].wait()
        scatters[n_chunks - 1].wait()

    return gather_kernel


def kernel(caption, table):
    B, L = caption.shape
    T = B * L
    idx = jnp.swapaxes(caption, 0, 1).reshape(T).astype(jnp.int32)
    table_p = jnp.pad(table, ((0, 0), (0, DP - D)))
    out = _build(T, table.shape[0])(table_p, idx)
    out64 = out[:, :D]
    return jnp.swapaxes(out64.reshape(L, B, D), 0, 1)
